# baseline (speedup 1.0000x reference)
import jax
import jax.numpy as jnp
from jax.experimental import pallas as pl
from jax.experimental.pallas import tpu as pltpu

B = 8
C_IN = 384
HW = 576
D = 64
K = 1024
BETA = 0.25


def _vq_body(z_ref, win_ref, bin_ref, cb_ref, wout_ref, bout_ref,
             out_ref, loss_ref):
    b = pl.program_id(0)
    zb = z_ref[0]
    zp = jnp.dot(win_ref[...], zb, preferred_element_type=jnp.float32)
    zp = zp + bin_ref[...]
    cb = cb_ref[...]
    cb2 = jnp.sum(cb * cb, axis=1, keepdims=True)
    dt = cb2 - 2.0 * jnp.dot(cb, zp, preferred_element_type=jnp.float32)
    minv = jnp.min(dt, axis=0, keepdims=True)
    iota = jax.lax.broadcasted_iota(jnp.int32, (K, HW), 0)
    idx = jnp.min(jnp.where(dt == minv, iota, K), axis=0, keepdims=True)
    onehot = (iota == idx).astype(jnp.float32)
    zq = jax.lax.dot_general(cb, onehot, (((0,), (0,)), ((), ())),
                             preferred_element_type=jnp.float32)
    ze2 = jnp.sum(zp * zp, axis=0, keepdims=True)
    part = jnp.sum(minv + ze2) * ((1.0 + BETA) / (B * HW * D))

    @pl.when(b == 0)
    def _init():
        loss_ref[...] = jnp.zeros_like(loss_ref)

    loss_ref[...] += part
    out = jnp.dot(wout_ref[...], zq, preferred_element_type=jnp.float32)
    out_ref[0] = out + bout_ref[...]


def kernel(z, W_in, b_in, codebook, W_out, b_out):
    z3 = z.reshape(B, C_IN, HW)
    out3, loss = pl.pallas_call(
        _vq_body,
        grid=(B,),
        in_specs=[
            pl.BlockSpec((1, C_IN, HW), lambda b: (b, 0, 0)),
            pl.BlockSpec((D, C_IN), lambda b: (0, 0)),
            pl.BlockSpec((D, 1), lambda b: (0, 0)),
            pl.BlockSpec((K, D), lambda b: (0, 0)),
            pl.BlockSpec((C_IN, D), lambda b: (0, 0)),
            pl.BlockSpec((C_IN, 1), lambda b: (0, 0)),
        ],
        out_specs=[
            pl.BlockSpec((1, C_IN, HW), lambda b: (b, 0, 0)),
            pl.BlockSpec((1, 1), lambda b: (0, 0)),
        ],
        out_shape=[
            jax.ShapeDtypeStruct((B, C_IN, HW), jnp.float32),
            jax.ShapeDtypeStruct((1, 1), jnp.float32),
        ],
    )(z3, W_in, b_in.reshape(D, 1), codebook, W_out, b_out.reshape(C_IN, 1))
    return loss[0, 0], out3.reshape(B, C_IN, 24, 24)


# speedup vs baseline: 1.0260x; 1.0260x over previous
import jax
import jax.numpy as jnp
from jax.experimental import pallas as pl
from jax.experimental.pallas import tpu as pltpu

B = 8
C_IN = 384
HW = 576
D = 64
K = 1024
BETA = 0.25


def _vq_body(z_ref, win_ref, bin_ref, cb_ref, wout_ref, bout_ref,
             out_ref, loss_ref, cba_ref):
    b = pl.program_id(0)
    cb = cb_ref[...]

    @pl.when(b == 0)
    def _prep():
        cba_ref[:, 0:D] = -2.0 * cb
        cba_ref[:, D:D + 1] = jnp.sum(cb * cb, axis=1, keepdims=True)

    zb = z_ref[0]
    zp_aug = jnp.dot(win_ref[...], zb, preferred_element_type=jnp.float32)
    zp_aug = zp_aug + bin_ref[...]
    dt = jnp.dot(cba_ref[...], zp_aug, preferred_element_type=jnp.float32)
    iota = jax.lax.broadcasted_iota(jnp.int32, (K, HW), 0)
    keyi = (jax.lax.bitcast_convert_type(dt, jnp.int32) & jnp.int32(-1024))
    keyf = jax.lax.bitcast_convert_type(keyi | iota, jnp.float32)
    kmin = jnp.min(keyf, axis=0, keepdims=True)
    onehot = (keyf == kmin).astype(jnp.float32)
    zq = jax.lax.dot_general(cb, onehot, (((0,), (0,)), ((), ())),
                             preferred_element_type=jnp.float32)
    minv = jax.lax.bitcast_convert_type(
        jax.lax.bitcast_convert_type(kmin, jnp.int32) & jnp.int32(-1024),
        jnp.float32)
    zp = zp_aug[0:D]
    ze2 = jnp.sum(zp * zp, axis=0, keepdims=True)
    part = jnp.sum(minv + ze2) * ((1.0 + BETA) / (B * HW * D))

    @pl.when(b == 0)
    def _init():
        loss_ref[...] = jnp.zeros_like(loss_ref)

    loss_ref[...] += part
    out = jnp.dot(wout_ref[...], zq, preferred_element_type=jnp.float32)
    out_ref[0] = out + bout_ref[...]


def kernel(z, W_in, b_in, codebook, W_out, b_out):
    z3 = z.reshape(B, C_IN, HW)
    win_aug = jnp.concatenate(
        [W_in, jnp.zeros((1, C_IN), jnp.float32)], axis=0)
    bin_aug = jnp.concatenate(
        [b_in, jnp.ones((1,), jnp.float32)], axis=0).reshape(D + 1, 1)
    out3, loss = pl.pallas_call(
        _vq_body,
        grid=(B,),
        in_specs=[
            pl.BlockSpec((1, C_IN, HW), lambda b: (b, 0, 0)),
            pl.BlockSpec((D + 1, C_IN), lambda b: (0, 0)),
            pl.BlockSpec((D + 1, 1), lambda b: (0, 0)),
            pl.BlockSpec((K, D), lambda b: (0, 0)),
            pl.BlockSpec((C_IN, D), lambda b: (0, 0)),
            pl.BlockSpec((C_IN, 1), lambda b: (0, 0)),
        ],
        out_specs=[
            pl.BlockSpec((1, C_IN, HW), lambda b: (b, 0, 0)),
            pl.BlockSpec((1, 1), lambda b: (0, 0)),
        ],
        out_shape=[
            jax.ShapeDtypeStruct((B, C_IN, HW), jnp.float32),
            jax.ShapeDtypeStruct((1, 1), jnp.float32),
        ],
        scratch_shapes=[pltpu.VMEM((K, D + 1), jnp.float32)],
    )(z3, win_aug, bin_aug, codebook, W_out, b_out.reshape(C_IN, 1))
    return loss[0, 0], out3.reshape(B, C_IN, 24, 24)


# speedup vs baseline: 1.0750x; 1.0477x over previous
import jax
import jax.numpy as jnp
from jax.experimental import pallas as pl
from jax.experimental.pallas import tpu as pltpu

B = 8
C_IN = 384
HW = 576
D = 64
K = 1024
BETA = 0.25
G = 4


def _vq_body(z_ref, win_ref, bin_ref, cb_ref, wout_ref, bout_ref,
             out_ref, loss_ref, cba_ref):
    b = pl.program_id(0)
    cb = cb_ref[...]

    @pl.when(b == 0)
    def _prep():
        cba_ref[:, 0:D] = -2.0 * cb
        cba_ref[:, D:D + 1] = jnp.sum(cb * cb, axis=1, keepdims=True)

    part = jnp.zeros((1, 1), jnp.float32)
    for g in range(G):
        zb = z_ref[g]
        zp_aug = jnp.dot(win_ref[...], zb,
                         preferred_element_type=jnp.float32)
        zp_aug = zp_aug + bin_ref[...]
        dt = jnp.dot(cba_ref[...], zp_aug,
                     preferred_element_type=jnp.float32)
        iota = jax.lax.broadcasted_iota(jnp.int32, (K, HW), 0)
        keyi = (jax.lax.bitcast_convert_type(dt, jnp.int32)
                & jnp.int32(-1024))
        keyf = jax.lax.bitcast_convert_type(keyi | iota, jnp.float32)
        kmin = jnp.min(keyf, axis=0, keepdims=True)
        onehot = (keyf == kmin).astype(jnp.float32)
        zq = jax.lax.dot_general(cb, onehot, (((0,), (0,)), ((), ())),
                                 preferred_element_type=jnp.float32)
        minv = jax.lax.bitcast_convert_type(
            jax.lax.bitcast_convert_type(kmin, jnp.int32) & jnp.int32(-1024),
            jnp.float32)
        zp = zp_aug[0:D]
        ze2 = jnp.sum(zp * zp, axis=0, keepdims=True)
        part = part + jnp.sum(minv + ze2) * ((1.0 + BETA) / (B * HW * D))
        out = jnp.dot(wout_ref[...], zq,
                      preferred_element_type=jnp.float32)
        out_ref[g] = out + bout_ref[...]

    @pl.when(b == 0)
    def _init():
        loss_ref[...] = jnp.zeros_like(loss_ref)

    loss_ref[...] += part


def kernel(z, W_in, b_in, codebook, W_out, b_out):
    z3 = z.reshape(B, C_IN, HW)
    win_aug = jnp.concatenate(
        [W_in, jnp.zeros((1, C_IN), jnp.float32)], axis=0)
    bin_aug = jnp.concatenate(
        [b_in, jnp.ones((1,), jnp.float32)], axis=0).reshape(D + 1, 1)
    out3, loss = pl.pallas_call(
        _vq_body,
        grid=(B // G,),
        in_specs=[
            pl.BlockSpec((G, C_IN, HW), lambda b: (b, 0, 0)),
            pl.BlockSpec((D + 1, C_IN), lambda b: (0, 0)),
            pl.BlockSpec((D + 1, 1), lambda b: (0, 0)),
            pl.BlockSpec((K, D), lambda b: (0, 0)),
            pl.BlockSpec((C_IN, D), lambda b: (0, 0)),
            pl.BlockSpec((C_IN, 1), lambda b: (0, 0)),
        ],
        out_specs=[
            pl.BlockSpec((G, C_IN, HW), lambda b: (b, 0, 0)),
            pl.BlockSpec((1, 1), lambda b: (0, 0)),
        ],
        out_shape=[
            jax.ShapeDtypeStruct((B, C_IN, HW), jnp.float32),
            jax.ShapeDtypeStruct((1, 1), jnp.float32),
        ],
        scratch_shapes=[pltpu.VMEM((K, D + 1), jnp.float32)],
    )(z3, win_aug, bin_aug, codebook, W_out, b_out.reshape(C_IN, 1))
    return loss[0, 0], out3.reshape(B, C_IN, 24, 24)
